# baseline (speedup 1.0000x reference)
import functools
import jax
import jax.numpy as jnp
import numpy as np
from jax import lax
from jax.experimental import pallas as pl
from jax.experimental.pallas import tpu as pltpu
from jax.experimental.pallas import tpu_sc as plsc

_ROWS = 8192
_COLS = 2048
_LANES = 16
_NV = _COLS // _LANES
_NW = 32
_ROWS_PER_W = _ROWS // _NW
_RBLK = 16
_NBLK = _ROWS_PER_W // _RBLK
_NPAIR = _NBLK // 2
_G = 16


def _scan_block(buf):
    row_iota = lax.iota(jnp.int32, _LANES)

    def step(i, carry_vec):
        off = i * _LANES
        for u in range(_G):
            v = buf[u, pl.ds(off, _LANES)]
            buf[u, pl.ds(off, _LANES)] = plsc.cumsum(v)
        for c in range(_LANES):
            col = jnp.full((_LANES,), off + c, jnp.int32)
            plsc.addupdate_scatter(buf, [row_iota, col], carry_vec)
        return plsc.load_gather(
            buf, [row_iota, jnp.full((_LANES,), off + _LANES - 1, jnp.int32)]
        )

    lax.fori_loop(0, _NV, step, jnp.zeros((_LANES,), jnp.float32), unroll=2)


def _sc_body(x_hbm, out_hbm, b0, b1, si0, si1, so0, so1):
    wid = lax.axis_index("s") * 2 + lax.axis_index("c")
    base = wid * _ROWS_PER_W

    def in_slice(b):
        return x_hbm.at[pl.ds(base + b * _RBLK, _RBLK)]

    def out_slice(b):
        return out_hbm.at[pl.ds(base + b * _RBLK, _RBLK)]

    pltpu.async_copy(in_slice(0), b0, si0)

    def body(k, c):
        blk = 2 * k

        @pl.when(k > 0)
        def _():
            pltpu.make_async_copy(b1, out_slice(blk - 1), so1).wait()

        pltpu.async_copy(in_slice(blk + 1), b1, si1)
        pltpu.make_async_copy(in_slice(blk), b0, si0).wait()
        _scan_block(b0)
        pltpu.async_copy(b0, out_slice(blk), so0)

        @pl.when(k < _NPAIR - 1)
        def _():
            pltpu.make_async_copy(b0, out_slice(blk), so0).wait()
            pltpu.async_copy(in_slice(blk + 2), b0, si0)

        pltpu.make_async_copy(in_slice(blk + 1), b1, si1).wait()
        _scan_block(b1)
        pltpu.async_copy(b1, out_slice(blk + 1), so1)
        return c

    lax.fori_loop(0, _NPAIR, body, 0, unroll=1)
    pltpu.make_async_copy(b0, out_slice(_NBLK - 2), so0).wait()
    pltpu.make_async_copy(b1, out_slice(_NBLK - 1), so1).wait()


@jax.jit
def kernel(x):
    mesh = plsc.VectorSubcoreMesh(core_axis_name="c", subcore_axis_name="s")
    run = pl.kernel(
        _sc_body,
        out_type=jax.ShapeDtypeStruct((_ROWS, _COLS), jnp.float32),
        mesh=mesh,
        scratch_types=[
            pltpu.VMEM((_RBLK, _COLS), jnp.float32),
            pltpu.VMEM((_RBLK, _COLS), jnp.float32),
            pltpu.SemaphoreType.DMA,
            pltpu.SemaphoreType.DMA,
            pltpu.SemaphoreType.DMA,
            pltpu.SemaphoreType.DMA,
        ],
        compiler_params=pltpu.CompilerParams(needs_layout_passes=False),
    )
    return run(x)


# speedup vs baseline: 6.1919x; 6.1919x over previous
import functools
import jax
import jax.numpy as jnp
import numpy as np
from jax import lax
from jax.experimental import pallas as pl
from jax.experimental.pallas import tpu as pltpu
from jax.experimental.pallas import tpu_sc as plsc

_ROWS = 8192
_COLS = 2048
_LANES = 16
_NV = _COLS // _LANES
_NW = 32
_ROWS_PER_W = _ROWS // _NW
_RBLK = 16
_NBLK = _ROWS_PER_W // _RBLK
_NPAIR = _NBLK // 2
_G = 16


def _scan_block(buf):
    def step(i, carries):
        off = i * _LANES
        svals = []
        for u in range(_G):
            v = buf[u, pl.ds(off, _LANES)]
            svals.append(plsc.cumsum(v))
        new = []
        for u in range(_G):
            buf[u, pl.ds(off, _LANES)] = svals[u] + carries[u]
            new.append(carries[u] + svals[u][_LANES - 1])
        return tuple(new)

    lax.fori_loop(0, _NV, step, (jnp.float32(0),) * _G, unroll=4)


def _sc_body(x_hbm, out_hbm, b0, b1, si0, si1, so0, so1):
    wid = lax.axis_index("s") * 2 + lax.axis_index("c")
    base = wid * _ROWS_PER_W

    def in_slice(b):
        return x_hbm.at[pl.ds(base + b * _RBLK, _RBLK)]

    def out_slice(b):
        return out_hbm.at[pl.ds(base + b * _RBLK, _RBLK)]

    pltpu.async_copy(in_slice(0), b0, si0)

    def body(k, c):
        blk = 2 * k

        @pl.when(k > 0)
        def _():
            pltpu.make_async_copy(b1, out_slice(blk - 1), so1).wait()

        pltpu.async_copy(in_slice(blk + 1), b1, si1)
        pltpu.make_async_copy(in_slice(blk), b0, si0).wait()
        _scan_block(b0)
        pltpu.async_copy(b0, out_slice(blk), so0)

        @pl.when(k < _NPAIR - 1)
        def _():
            pltpu.make_async_copy(b0, out_slice(blk), so0).wait()
            pltpu.async_copy(in_slice(blk + 2), b0, si0)

        pltpu.make_async_copy(in_slice(blk + 1), b1, si1).wait()
        _scan_block(b1)
        pltpu.async_copy(b1, out_slice(blk + 1), so1)
        return c

    lax.fori_loop(0, _NPAIR, body, 0, unroll=1)
    pltpu.make_async_copy(b0, out_slice(_NBLK - 2), so0).wait()
    pltpu.make_async_copy(b1, out_slice(_NBLK - 1), so1).wait()


@jax.jit
def kernel(x):
    mesh = plsc.VectorSubcoreMesh(core_axis_name="c", subcore_axis_name="s")
    run = pl.kernel(
        _sc_body,
        out_type=jax.ShapeDtypeStruct((_ROWS, _COLS), jnp.float32),
        mesh=mesh,
        scratch_types=[
            pltpu.VMEM((_RBLK, _COLS), jnp.float32),
            pltpu.VMEM((_RBLK, _COLS), jnp.float32),
            pltpu.SemaphoreType.DMA,
            pltpu.SemaphoreType.DMA,
            pltpu.SemaphoreType.DMA,
            pltpu.SemaphoreType.DMA,
        ],
        compiler_params=pltpu.CompilerParams(needs_layout_passes=False),
    )
    return run(x)


# speedup vs baseline: 6.1959x; 1.0006x over previous
import functools
import jax
import jax.numpy as jnp
import numpy as np
from jax import lax
from jax.experimental import pallas as pl
from jax.experimental.pallas import tpu as pltpu
from jax.experimental.pallas import tpu_sc as plsc

_ROWS = 8192
_COLS = 2048
_LANES = 16
_NV = _COLS // _LANES
_NW = 32
_ROWS_PER_W = _ROWS // _NW
_RBLK = 16
_NBLK = _ROWS_PER_W // _RBLK
_NPAIR = _NBLK // 2
_G = 16


def _scan_block(buf):
    @plsc.parallel_loop(0, _NV, carry=(jnp.float32(0),) * _G, unroll=4)
    def _loop(i, carries):
        off = i * _LANES
        svals = []
        for u in range(_G):
            v = buf[u, pl.ds(off, _LANES)]
            svals.append(plsc.cumsum(v))
        new = []
        for u in range(_G):
            buf[u, pl.ds(off, _LANES)] = svals[u] + carries[u]
            new.append(carries[u] + svals[u][_LANES - 1])
        return tuple(new)


def _sc_body(x_hbm, out_hbm, b0, b1, si0, si1, so0, so1):
    wid = lax.axis_index("s") * 2 + lax.axis_index("c")
    base = wid * _ROWS_PER_W

    def in_slice(b):
        return x_hbm.at[pl.ds(base + b * _RBLK, _RBLK)]

    def out_slice(b):
        return out_hbm.at[pl.ds(base + b * _RBLK, _RBLK)]

    pltpu.async_copy(in_slice(0), b0, si0)

    def body(k, c):
        blk = 2 * k

        @pl.when(k > 0)
        def _():
            pltpu.make_async_copy(b1, out_slice(blk - 1), so1).wait()

        pltpu.async_copy(in_slice(blk + 1), b1, si1)
        pltpu.make_async_copy(in_slice(blk), b0, si0).wait()
        _scan_block(b0)
        pltpu.async_copy(b0, out_slice(blk), so0)

        @pl.when(k < _NPAIR - 1)
        def _():
            pltpu.make_async_copy(b0, out_slice(blk), so0).wait()
            pltpu.async_copy(in_slice(blk + 2), b0, si0)

        pltpu.make_async_copy(in_slice(blk + 1), b1, si1).wait()
        _scan_block(b1)
        pltpu.async_copy(b1, out_slice(blk + 1), so1)
        return c

    lax.fori_loop(0, _NPAIR, body, 0, unroll=1)
    pltpu.make_async_copy(b0, out_slice(_NBLK - 2), so0).wait()
    pltpu.make_async_copy(b1, out_slice(_NBLK - 1), so1).wait()


@jax.jit
def kernel(x):
    mesh = plsc.VectorSubcoreMesh(core_axis_name="c", subcore_axis_name="s")
    run = pl.kernel(
        _sc_body,
        out_type=jax.ShapeDtypeStruct((_ROWS, _COLS), jnp.float32),
        mesh=mesh,
        scratch_types=[
            pltpu.VMEM((_RBLK, _COLS), jnp.float32),
            pltpu.VMEM((_RBLK, _COLS), jnp.float32),
            pltpu.SemaphoreType.DMA,
            pltpu.SemaphoreType.DMA,
            pltpu.SemaphoreType.DMA,
            pltpu.SemaphoreType.DMA,
        ],
        compiler_params=pltpu.CompilerParams(needs_layout_passes=False),
    )
    return run(x)


# speedup vs baseline: 8.0611x; 1.3010x over previous
import functools
import jax
import jax.numpy as jnp
import numpy as np
from jax import lax
from jax.experimental import pallas as pl
from jax.experimental.pallas import tpu as pltpu
from jax.experimental.pallas import tpu_sc as plsc

_ROWS = 8192
_COLS = 2048
_LANES = 16
_NV = _COLS // _LANES
_NW = 32
_ROWS_PER_W = _ROWS // _NW
_RBLK = 8
_NBLK = _ROWS_PER_W // _RBLK
_NPAIR = _NBLK // 2
_G = _RBLK


def _scan_block(src, dst):
    @plsc.parallel_loop(0, _NV, carry=(jnp.float32(0),) * _G, unroll=4)
    def _loop(i, carries):
        off = i * _LANES
        svals = []
        for u in range(_G):
            v = src[u, pl.ds(off, _LANES)]
            svals.append(plsc.cumsum(v))
        new = []
        for u in range(_G):
            dst[u, pl.ds(off, _LANES)] = svals[u] + carries[u]
            new.append(carries[u] + svals[u][_LANES - 1])
        return tuple(new)


def _sc_body(x_hbm, out_hbm, in0, in1, ou0, ou1, si0, si1, so0, so1):
    wid = lax.axis_index("s") * 2 + lax.axis_index("c")
    base = wid * _ROWS_PER_W

    def in_slice(b):
        return x_hbm.at[pl.ds(base + b * _RBLK, _RBLK)]

    def out_slice(b):
        return out_hbm.at[pl.ds(base + b * _RBLK, _RBLK)]

    pltpu.async_copy(in_slice(0), in0, si0)

    def body(k, c):
        b0 = 2 * k
        pltpu.async_copy(in_slice(b0 + 1), in1, si1)
        pltpu.make_async_copy(in_slice(b0), in0, si0).wait()

        @pl.when(k > 0)
        def _():
            pltpu.make_async_copy(ou0, out_slice(b0), so0).wait()

        _scan_block(in0, ou0)
        pltpu.async_copy(ou0, out_slice(b0), so0)

        @pl.when(k < _NPAIR - 1)
        def _():
            pltpu.async_copy(in_slice(b0 + 2), in0, si0)

        pltpu.make_async_copy(in_slice(b0 + 1), in1, si1).wait()

        @pl.when(k > 0)
        def _():
            pltpu.make_async_copy(ou1, out_slice(b0 + 1), so1).wait()

        _scan_block(in1, ou1)
        pltpu.async_copy(ou1, out_slice(b0 + 1), so1)
        return c

    lax.fori_loop(0, _NPAIR, body, 0, unroll=1)
    pltpu.make_async_copy(ou0, out_slice(_NBLK - 2), so0).wait()
    pltpu.make_async_copy(ou1, out_slice(_NBLK - 1), so1).wait()


@jax.jit
def kernel(x):
    mesh = plsc.VectorSubcoreMesh(core_axis_name="c", subcore_axis_name="s")
    run = pl.kernel(
        _sc_body,
        out_type=jax.ShapeDtypeStruct((_ROWS, _COLS), jnp.float32),
        mesh=mesh,
        scratch_types=[
            pltpu.VMEM((_RBLK, _COLS), jnp.float32),
            pltpu.VMEM((_RBLK, _COLS), jnp.float32),
            pltpu.VMEM((_RBLK, _COLS), jnp.float32),
            pltpu.VMEM((_RBLK, _COLS), jnp.float32),
            pltpu.SemaphoreType.DMA,
            pltpu.SemaphoreType.DMA,
            pltpu.SemaphoreType.DMA,
            pltpu.SemaphoreType.DMA,
        ],
        compiler_params=pltpu.CompilerParams(needs_layout_passes=False),
    )
    return run(x)


# speedup vs baseline: 8.3896x; 1.0408x over previous
import functools
import jax
import jax.numpy as jnp
import numpy as np
from jax import lax
from jax.experimental import pallas as pl
from jax.experimental.pallas import tpu as pltpu
from jax.experimental.pallas import tpu_sc as plsc

_ROWS = 8192
_COLS = 2048
_LANES = 16
_NV = _COLS // _LANES
_NW = 32
_ROWS_PER_W = _ROWS // _NW
_RBLK = 8
_NBLK = _ROWS_PER_W // _RBLK
_NPAIR = _NBLK // 2
_G = _RBLK


def _scan_block(src, dst):
    @plsc.parallel_loop(0, _NV, carry=(jnp.float32(0),) * _G, unroll=8)
    def _loop(i, carries):
        off = i * _LANES
        svals = []
        for u in range(_G):
            v = src[u, pl.ds(off, _LANES)]
            svals.append(plsc.cumsum(v))
        new = []
        for u in range(_G):
            dst[u, pl.ds(off, _LANES)] = svals[u] + carries[u]
            new.append(carries[u] + svals[u][_LANES - 1])
        return tuple(new)


def _sc_body(x_hbm, out_hbm, in0, in1, ou0, ou1, si0, si1, so0, so1):
    wid = lax.axis_index("s") * 2 + lax.axis_index("c")
    base = wid * _ROWS_PER_W

    def in_slice(b):
        return x_hbm.at[pl.ds(base + b * _RBLK, _RBLK)]

    def out_slice(b):
        return out_hbm.at[pl.ds(base + b * _RBLK, _RBLK)]

    pltpu.async_copy(in_slice(0), in0, si0)

    def body(k, c):
        b0 = 2 * k
        pltpu.async_copy(in_slice(b0 + 1), in1, si1)
        pltpu.make_async_copy(in_slice(b0), in0, si0).wait()

        @pl.when(k > 0)
        def _():
            pltpu.make_async_copy(ou0, out_slice(b0), so0).wait()

        _scan_block(in0, ou0)
        pltpu.async_copy(ou0, out_slice(b0), so0)

        @pl.when(k < _NPAIR - 1)
        def _():
            pltpu.async_copy(in_slice(b0 + 2), in0, si0)

        pltpu.make_async_copy(in_slice(b0 + 1), in1, si1).wait()

        @pl.when(k > 0)
        def _():
            pltpu.make_async_copy(ou1, out_slice(b0 + 1), so1).wait()

        _scan_block(in1, ou1)
        pltpu.async_copy(ou1, out_slice(b0 + 1), so1)
        return c

    lax.fori_loop(0, _NPAIR, body, 0, unroll=1)
    pltpu.make_async_copy(ou0, out_slice(_NBLK - 2), so0).wait()
    pltpu.make_async_copy(ou1, out_slice(_NBLK - 1), so1).wait()


@jax.jit
def kernel(x):
    mesh = plsc.VectorSubcoreMesh(core_axis_name="c", subcore_axis_name="s")
    run = pl.kernel(
        _sc_body,
        out_type=jax.ShapeDtypeStruct((_ROWS, _COLS), jnp.float32),
        mesh=mesh,
        scratch_types=[
            pltpu.VMEM((_RBLK, _COLS), jnp.float32),
            pltpu.VMEM((_RBLK, _COLS), jnp.float32),
            pltpu.VMEM((_RBLK, _COLS), jnp.float32),
            pltpu.VMEM((_RBLK, _COLS), jnp.float32),
            pltpu.SemaphoreType.DMA,
            pltpu.SemaphoreType.DMA,
            pltpu.SemaphoreType.DMA,
            pltpu.SemaphoreType.DMA,
        ],
        compiler_params=pltpu.CompilerParams(needs_layout_passes=False),
    )
    return run(x)
